# baseline (speedup 1.0000x reference)
import functools

import jax
import jax.numpy as jnp
from jax import lax
from jax.experimental import pallas as pl
from jax.experimental.pallas import tpu as pltpu
from jax.experimental.pallas import tpu_sc as plsc

EMBED_DIM = 32
BATCH = 16384
LANES = 16
PACK = 128 // EMBED_DIM

N_USER = 1000000
N_ITEM = 100000

NUM_CORES = 2
NUM_SUBCORES = 16
NUM_WORKERS = NUM_CORES * NUM_SUBCORES
B_PER_W = BATCH // NUM_WORKERS
CHUNK = 128
HALF = B_PER_W // 2
N_CHUNK = HALF // CHUNK

U_BLOCKS = N_USER // CHUNK
I_BLOCKS = N_ITEM // CHUNK
U_TAIL = N_USER - U_BLOCKS * CHUNK
I_TAIL = N_ITEM - I_BLOCKS * CHUNK
U_LINES = N_USER // PACK
I_LINES = N_ITEM // PACK

_MESH = plsc.VectorSubcoreMesh(core_axis_name="c", subcore_axis_name="s")
_PARAMS = pltpu.CompilerParams(needs_layout_passes=False)


def _transpose_block(colbuf, outbuf, lane):
    for l in range(CHUNK // PACK):
        for m in range(PACK):
            col = jnp.full((LANES,), PACK * l + m, jnp.int32)
            for o in range(EMBED_DIM // LANES):
                d_vec = o * LANES + lane
                vals = plsc.load_gather(colbuf, [d_vec, col])
                outbuf[l, pl.ds(m * EMBED_DIM + o * LANES, LANES)] = vals


def _relayout_body(utab_t, itab_t, utail, itail, ucomp, icomp,
                   colbuf, outbuf, tailbuf, sem):
    wid = lax.axis_index("s") * NUM_CORES + lax.axis_index("c")
    lane = lax.iota(jnp.int32, LANES)

    def make_loop(src, dst, n_blocks):
        def body(k, _):
            g = k * NUM_WORKERS + wid

            @pl.when(g < n_blocks)
            def _():
                pltpu.sync_copy(src.at[:, pl.ds(g * CHUNK, CHUNK)], colbuf)
                _transpose_block(colbuf, outbuf, lane)
                pltpu.sync_copy(
                    outbuf, dst.at[pl.ds(g * (CHUNK // PACK), CHUNK // PACK)])
            return 0
        return body

    u_iters = (U_BLOCKS + NUM_WORKERS - 1) // NUM_WORKERS
    i_iters = (I_BLOCKS + NUM_WORKERS - 1) // NUM_WORKERS
    lax.fori_loop(0, u_iters, make_loop(utab_t, ucomp, U_BLOCKS), 0)
    lax.fori_loop(0, i_iters, make_loop(itab_t, icomp, I_BLOCKS), 0)

    @pl.when(wid == 0)
    def _():
        pltpu.sync_copy(utail, tailbuf)
        pltpu.sync_copy(
            tailbuf.at[pl.ds(0, U_TAIL // PACK)],
            ucomp.at[pl.ds(U_BLOCKS * (CHUNK // PACK), U_TAIL // PACK)])
        pltpu.sync_copy(itail, tailbuf.at[pl.ds(0, I_TAIL // PACK)])
        pltpu.sync_copy(
            tailbuf.at[pl.ds(0, I_TAIL // PACK)],
            icomp.at[pl.ds(I_BLOCKS * (CHUNK // PACK), I_TAIL // PACK)])


_relayout_call = functools.partial(
    pl.kernel,
    mesh=_MESH,
    out_type=(jax.ShapeDtypeStruct((U_LINES, CHUNK), jnp.float32),
              jax.ShapeDtypeStruct((I_LINES, CHUNK), jnp.float32)),
    compiler_params=_PARAMS,
    scratch_types=[
        pltpu.VMEM((EMBED_DIM, CHUNK), jnp.float32),
        pltpu.VMEM((CHUNK // PACK, CHUNK), jnp.float32),
        pltpu.VMEM((U_TAIL // PACK, CHUNK), jnp.float32),
        pltpu.SemaphoreType.DMA,
    ],
)(_relayout_body)


def _dot_body(uids_hbm, iids_hbm, utab_hbm, itab_hbm, out_hbm,
              uid_v, iid_v, ulid_v, ilid_v, ulines, ilines, out_v, sem):
    wid = lax.axis_index("s") * NUM_CORES + lax.axis_index("c")
    base = wid * B_PER_W
    idx_row = wid * (B_PER_W // CHUNK)

    pltpu.sync_copy(uids_hbm.at[pl.ds(idx_row, B_PER_W // CHUNK)], uid_v)
    pltpu.sync_copy(iids_hbm.at[pl.ds(idx_row, B_PER_W // CHUNK)], iid_v)

    for j in range(B_PER_W // CHUNK):
        for k in range(CHUNK // LANES):
            s = pl.ds(k * LANES, LANES)
            ulid_v[j, s] = jax.lax.shift_right_logical(uid_v[j, s], 2)
            ilid_v[j, s] = jax.lax.shift_right_logical(iid_v[j, s], 2)

    lane = lax.iota(jnp.int32, LANES)

    for h in range(2):
        copies = []
        for j in range(N_CHUNK):
            jj = h * N_CHUNK + j
            copies.append(pltpu.async_copy(
                utab_hbm.at[ulid_v.at[jj]],
                ulines.at[pl.ds(j * CHUNK, CHUNK)], sem))
            copies.append(pltpu.async_copy(
                itab_hbm.at[ilid_v.at[jj]],
                ilines.at[pl.ds(j * CHUNK, CHUNK)], sem))
        for c in copies:
            c.wait()

        def body(g, _):
            b0 = g * LANES
            rows = b0 + lane
            jj = h * N_CHUNK + b0 // CHUNK
            s = pl.ds(b0 % CHUNK, LANES)
            uoff = (uid_v[jj, s] & (PACK - 1)) * EMBED_DIM
            ioff = (iid_v[jj, s] & (PACK - 1)) * EMBED_DIM
            acc = jnp.zeros((LANES,), jnp.float32)
            for d in range(EMBED_DIM):
                uc = plsc.load_gather(ulines, [rows, uoff + d])
                vc = plsc.load_gather(ilines, [rows, ioff + d])
                acc = acc + uc * vc
            out_v[pl.ds(h * HALF + b0, LANES)] = acc
            return 0

        lax.fori_loop(0, HALF // LANES, body, 0)

    pltpu.sync_copy(out_v, out_hbm.at[pl.ds(base, B_PER_W)])


_gather_call = functools.partial(
    pl.kernel,
    mesh=_MESH,
    out_type=jax.ShapeDtypeStruct((BATCH,), jnp.float32),
    compiler_params=_PARAMS,
    scratch_types=[
        pltpu.VMEM((B_PER_W // CHUNK, CHUNK), jnp.int32),
        pltpu.VMEM((B_PER_W // CHUNK, CHUNK), jnp.int32),
        pltpu.VMEM((B_PER_W // CHUNK, CHUNK), jnp.int32),
        pltpu.VMEM((B_PER_W // CHUNK, CHUNK), jnp.int32),
        pltpu.VMEM((HALF, CHUNK), jnp.float32),
        pltpu.VMEM((HALF, CHUNK), jnp.float32),
        pltpu.VMEM((B_PER_W,), jnp.float32),
        pltpu.SemaphoreType.DMA,
    ],
)(_dot_body)


@jax.jit
def kernel(user_ids, item_ids, user_table, item_table):
    uids = jnp.asarray(user_ids, jnp.int32).reshape(BATCH // CHUNK, CHUNK)
    iids = jnp.asarray(item_ids, jnp.int32).reshape(BATCH // CHUNK, CHUNK)
    utail = user_table[U_BLOCKS * CHUNK:].reshape(U_TAIL // PACK, CHUNK)
    itail = item_table[I_BLOCKS * CHUNK:].reshape(I_TAIL // PACK, CHUNK)
    ucomp, icomp = _relayout_call(
        user_table.T, item_table.T, utail, itail)
    return _gather_call(uids, iids, ucomp, icomp)


# speedup vs baseline: 1.1243x; 1.1243x over previous
import functools

import jax
import jax.numpy as jnp
from jax import lax
from jax.experimental import pallas as pl
from jax.experimental.pallas import tpu as pltpu
from jax.experimental.pallas import tpu_sc as plsc

EMBED_DIM = 32
BATCH = 16384
LANES = 16
PACK = 128 // EMBED_DIM

N_USER = 1000000
N_ITEM = 100000

NUM_CORES = 2
NUM_SUBCORES = 16
NUM_WORKERS = NUM_CORES * NUM_SUBCORES
B_PER_W = BATCH // NUM_WORKERS
CHUNK = 128
HALF = B_PER_W // 2
N_CHUNK = HALF // CHUNK

WBLK = 512
WLINES = WBLK // PACK
U_BLOCKS = N_USER // WBLK
I_BLOCKS = N_ITEM // WBLK
U_TAIL = N_USER - U_BLOCKS * WBLK
I_TAIL = N_ITEM - I_BLOCKS * WBLK
U_LINES = N_USER // PACK
I_LINES = N_ITEM // PACK

_MESH = plsc.VectorSubcoreMesh(core_axis_name="c", subcore_axis_name="s")
_PARAMS = pltpu.CompilerParams(needs_layout_passes=False)


def _transpose_block(colbuf, outbuf, lane):
    def line(l, _):
        for m in range(PACK):
            col = jnp.full((LANES,), m, jnp.int32) + PACK * l
            for o in range(EMBED_DIM // LANES):
                d_vec = o * LANES + lane
                vals = plsc.load_gather(colbuf, [d_vec, col])
                outbuf[l, pl.ds(m * EMBED_DIM + o * LANES, LANES)] = vals
        return 0

    lax.fori_loop(0, WLINES, line, 0)


def _relayout_body(utab_t, itab_t, utail, itail, ucomp, icomp,
                   colbuf, outbuf, tailbuf, sem):
    wid = lax.axis_index("s") * NUM_CORES + lax.axis_index("c")
    lane = lax.iota(jnp.int32, LANES)

    def make_loop(src, dst, n_blocks):
        def body(k, _):
            g = k * NUM_WORKERS + wid

            @pl.when(g < n_blocks)
            def _():
                pltpu.sync_copy(src.at[:, pl.ds(g * WBLK, WBLK)], colbuf)
                _transpose_block(colbuf, outbuf, lane)
                pltpu.sync_copy(outbuf, dst.at[pl.ds(g * WLINES, WLINES)])
            return 0
        return body

    u_iters = (U_BLOCKS + NUM_WORKERS - 1) // NUM_WORKERS
    i_iters = (I_BLOCKS + NUM_WORKERS - 1) // NUM_WORKERS
    lax.fori_loop(0, u_iters, make_loop(utab_t, ucomp, U_BLOCKS), 0)
    lax.fori_loop(0, i_iters, make_loop(itab_t, icomp, I_BLOCKS), 0)

    @pl.when(wid == 0)
    def _():
        pltpu.sync_copy(utail, tailbuf.at[pl.ds(0, U_TAIL // PACK)])
        pltpu.sync_copy(
            tailbuf.at[pl.ds(0, U_TAIL // PACK)],
            ucomp.at[pl.ds(U_BLOCKS * WLINES, U_TAIL // PACK)])
        pltpu.sync_copy(itail, tailbuf.at[pl.ds(0, I_TAIL // PACK)])
        pltpu.sync_copy(
            tailbuf.at[pl.ds(0, I_TAIL // PACK)],
            icomp.at[pl.ds(I_BLOCKS * WLINES, I_TAIL // PACK)])


_relayout_call = functools.partial(
    pl.kernel,
    mesh=_MESH,
    out_type=(jax.ShapeDtypeStruct((U_LINES, CHUNK), jnp.float32),
              jax.ShapeDtypeStruct((I_LINES, CHUNK), jnp.float32)),
    compiler_params=_PARAMS,
    scratch_types=[
        pltpu.VMEM((EMBED_DIM, WBLK), jnp.float32),
        pltpu.VMEM((WLINES, CHUNK), jnp.float32),
        pltpu.VMEM((I_TAIL // PACK, CHUNK), jnp.float32),
        pltpu.SemaphoreType.DMA,
    ],
)(_relayout_body)


def _dot_body(uids_hbm, iids_hbm, utab_hbm, itab_hbm, out_hbm,
              uid_v, iid_v, ulid_v, ilid_v, ulines, ilines, out_v, sem):
    wid = lax.axis_index("s") * NUM_CORES + lax.axis_index("c")
    base = wid * B_PER_W
    idx_row = wid * (B_PER_W // CHUNK)

    pltpu.sync_copy(uids_hbm.at[pl.ds(idx_row, B_PER_W // CHUNK)], uid_v)
    pltpu.sync_copy(iids_hbm.at[pl.ds(idx_row, B_PER_W // CHUNK)], iid_v)

    for j in range(B_PER_W // CHUNK):
        for k in range(CHUNK // LANES):
            s = pl.ds(k * LANES, LANES)
            ulid_v[j, s] = jax.lax.shift_right_logical(uid_v[j, s], 2)
            ilid_v[j, s] = jax.lax.shift_right_logical(iid_v[j, s], 2)

    lane = lax.iota(jnp.int32, LANES)

    for h in range(2):
        copies = []
        for j in range(N_CHUNK):
            jj = h * N_CHUNK + j
            copies.append(pltpu.async_copy(
                utab_hbm.at[ulid_v.at[jj]],
                ulines.at[pl.ds(j * CHUNK, CHUNK)], sem))
            copies.append(pltpu.async_copy(
                itab_hbm.at[ilid_v.at[jj]],
                ilines.at[pl.ds(j * CHUNK, CHUNK)], sem))
        for c in copies:
            c.wait()

        def body(g, _):
            b0 = g * LANES
            rows = b0 + lane
            jj = h * N_CHUNK + b0 // CHUNK
            s = pl.ds(b0 % CHUNK, LANES)
            uoff = (uid_v[jj, s] & (PACK - 1)) * EMBED_DIM
            ioff = (iid_v[jj, s] & (PACK - 1)) * EMBED_DIM
            acc = jnp.zeros((LANES,), jnp.float32)
            for d in range(EMBED_DIM):
                uc = plsc.load_gather(ulines, [rows, uoff + d])
                vc = plsc.load_gather(ilines, [rows, ioff + d])
                acc = acc + uc * vc
            out_v[pl.ds(h * HALF + b0, LANES)] = acc
            return 0

        lax.fori_loop(0, HALF // LANES, body, 0)

    pltpu.sync_copy(out_v, out_hbm.at[pl.ds(base, B_PER_W)])


_gather_call = functools.partial(
    pl.kernel,
    mesh=_MESH,
    out_type=jax.ShapeDtypeStruct((BATCH,), jnp.float32),
    compiler_params=_PARAMS,
    scratch_types=[
        pltpu.VMEM((B_PER_W // CHUNK, CHUNK), jnp.int32),
        pltpu.VMEM((B_PER_W // CHUNK, CHUNK), jnp.int32),
        pltpu.VMEM((B_PER_W // CHUNK, CHUNK), jnp.int32),
        pltpu.VMEM((B_PER_W // CHUNK, CHUNK), jnp.int32),
        pltpu.VMEM((HALF, CHUNK), jnp.float32),
        pltpu.VMEM((HALF, CHUNK), jnp.float32),
        pltpu.VMEM((B_PER_W,), jnp.float32),
        pltpu.SemaphoreType.DMA,
    ],
)(_dot_body)


@jax.jit
def kernel(user_ids, item_ids, user_table, item_table):
    uids = jnp.asarray(user_ids, jnp.int32).reshape(BATCH // CHUNK, CHUNK)
    iids = jnp.asarray(item_ids, jnp.int32).reshape(BATCH // CHUNK, CHUNK)
    utail = user_table[U_BLOCKS * WBLK:].reshape(U_TAIL // PACK, CHUNK)
    itail = item_table[I_BLOCKS * WBLK:].reshape(I_TAIL // PACK, CHUNK)
    ucomp, icomp = _relayout_call(
        user_table.T, item_table.T, utail, itail)
    return _gather_call(uids, iids, ucomp, icomp)


# speedup vs baseline: 2.1206x; 1.8861x over previous
import functools

import jax
import jax.numpy as jnp
from jax import lax
from jax.experimental import pallas as pl
from jax.experimental.pallas import tpu as pltpu
from jax.experimental.pallas import tpu_sc as plsc

EMBED_DIM = 32
BATCH = 16384
LANES = 16

NUM_CORES = 2
NUM_SUBCORES = 16
NUM_WORKERS = NUM_CORES * NUM_SUBCORES
B_PER_W = BATCH // NUM_WORKERS
CHUNK = 128
N_CHUNK = B_PER_W // CHUNK


def _dot_body(uids_hbm, iids_hbm, utab_hbm, itab_hbm, out_hbm,
              uid_v, iid_v, urows, irows, out_v, sem):
    wid = lax.axis_index("s") * NUM_CORES + lax.axis_index("c")
    base = wid * B_PER_W
    idx_row = wid * N_CHUNK

    pltpu.sync_copy(uids_hbm.at[pl.ds(idx_row, N_CHUNK)], uid_v)
    pltpu.sync_copy(iids_hbm.at[pl.ds(idx_row, N_CHUNK)], iid_v)

    copies = []
    for j in range(N_CHUNK):
        copies.append(pltpu.async_copy(
            utab_hbm.at[uid_v.at[j]], urows.at[pl.ds(j * CHUNK, CHUNK)], sem))
        copies.append(pltpu.async_copy(
            itab_hbm.at[iid_v.at[j]], irows.at[pl.ds(j * CHUNK, CHUNK)], sem))
    for c in copies:
        c.wait()

    lane = lax.iota(jnp.int32, LANES)

    def body(g, _):
        rows = g * LANES + lane
        acc = jnp.zeros((LANES,), jnp.float32)
        for d in range(EMBED_DIM):
            col = jnp.full((LANES,), d, jnp.int32)
            uc = plsc.load_gather(urows, [rows, col])
            vc = plsc.load_gather(irows, [rows, col])
            acc = acc + uc * vc
        out_v[pl.ds(g * LANES, LANES)] = acc
        return 0

    lax.fori_loop(0, B_PER_W // LANES, body, 0)

    pltpu.sync_copy(out_v, out_hbm.at[pl.ds(base, B_PER_W)])


_sc_call = functools.partial(
    pl.kernel,
    mesh=plsc.VectorSubcoreMesh(core_axis_name="c", subcore_axis_name="s"),
    out_type=jax.ShapeDtypeStruct((BATCH,), jnp.float32),
    compiler_params=pltpu.CompilerParams(
        needs_layout_passes=False, use_tc_tiling_on_sc=False),
    scratch_types=[
        pltpu.VMEM((N_CHUNK, CHUNK), jnp.int32),
        pltpu.VMEM((N_CHUNK, CHUNK), jnp.int32),
        pltpu.VMEM((B_PER_W, EMBED_DIM), jnp.float32),
        pltpu.VMEM((B_PER_W, EMBED_DIM), jnp.float32),
        pltpu.VMEM((B_PER_W,), jnp.float32),
        pltpu.SemaphoreType.DMA,
    ],
)(_dot_body)


@jax.jit
def kernel(user_ids, item_ids, user_table, item_table):
    uids = jnp.asarray(user_ids, jnp.int32).reshape(NUM_WORKERS * N_CHUNK, CHUNK)
    iids = jnp.asarray(item_ids, jnp.int32).reshape(NUM_WORKERS * N_CHUNK, CHUNK)
    return _sc_call(uids, iids, user_table, item_table)
